# Initial kernel scaffold; baseline (speedup 1.0000x reference)
import jax
import jax.numpy as jnp
from jax.experimental import pallas as pl


def kernel(x, edge_attr, edge_index, Wm0, bm0, Wa0, ba0, Wm1, bm1, Wa1, ba1, We0, be0):
    raise NotImplementedError("write your pallas kernel here")


# speedup vs baseline: 1.2874x; 1.2874x over previous
import functools

import jax
import jax.numpy as jnp
from jax import lax
from jax.experimental import pallas as pl
from jax.experimental.pallas import tpu as pltpu
from jax.experimental.pallas import tpu_sc as plsc

N = 10000
E = 160000
D = 256
DE = 16
HD = 128

NC = 2
NS = 16
LANES = 16

NP = 10240
SLAB = NP // NS

EPT = E // NS
CH = 80
NCHUNK = EPT // CH

NW = NC * NS
EPW = E // NW
CH2 = 40
NCHUNK2 = EPW // CH2

_mesh = plsc.VectorSubcoreMesh(core_axis_name="c", subcore_axis_name="s")


@functools.partial(
    pl.kernel,
    out_type=jax.ShapeDtypeStruct((2 * NP, HD), jnp.float32),
    mesh=_mesh,
    scratch_types=[
        pltpu.VMEM((CH,), jnp.int32),
        pltpu.VMEM((CH,), jnp.int32),
        pltpu.VMEM((CH, HD), jnp.float32),
        pltpu.VMEM((CH, HD), jnp.float32),
        pltpu.VMEM_SHARED((NP, HD), jnp.float32),
        pltpu.SemaphoreType.DMA,
    ],
)
def _sc_scatter(h_hbm, c_hbm, src_hbm, dst_hbm, s_out,
                sidx, didx, rows, crows, acc, sem):
    cid = lax.axis_index("c")
    tid = lax.axis_index("s")

    def zrow(i, _):
        for j in range(HD // LANES):
            rows[i, pl.ds(j * LANES, LANES)] = jnp.zeros((LANES,), jnp.float32)
        return 0
    lax.fori_loop(0, CH, zrow, 0)
    for k in range(SLAB // CH):
        pltpu.sync_copy(rows, acc.at[pl.ds(tid * SLAB + k * CH, CH)])

    plsc.subcore_barrier()

    ebase = tid * EPT

    def step(k, _):
        e0 = ebase + k * CH
        pltpu.sync_copy(src_hbm.at[pl.ds(cid * E + e0, CH)], sidx)
        pltpu.sync_copy(dst_hbm.at[pl.ds(e0, CH)], didx)
        pltpu.async_copy(h_hbm.at[sidx], rows, sem).wait()
        pltpu.sync_copy(c_hbm.at[pl.ds(cid * E + e0, CH)], crows)

        def rfix(i, _):
            for j in range(HD // LANES):
                sl = pl.ds(j * LANES, LANES)
                rows[i, sl] = jnp.maximum(rows[i, sl] + crows[i, sl], 0.0)
            return 0
        lax.fori_loop(0, CH, rfix, 0)

        pltpu.sync_copy(rows, acc.at[didx], add=True)
        return 0
    lax.fori_loop(0, NCHUNK, step, 0)

    plsc.subcore_barrier()

    for k in range(SLAB // CH):
        r0 = tid * SLAB + k * CH
        pltpu.sync_copy(acc.at[pl.ds(r0, CH)], rows)
        pltpu.sync_copy(rows, s_out.at[pl.ds(cid * NP + r0, CH)])


@functools.partial(
    pl.kernel,
    out_type=jax.ShapeDtypeStruct((2 * NP, HD), jnp.float32),
    mesh=_mesh,
    scratch_types=[
        pltpu.VMEM((CH2,), jnp.int32),
        pltpu.VMEM((CH2, HD), jnp.float32),
        pltpu.VMEM_SHARED((NP, HD), jnp.float32),
        pltpu.SemaphoreType.DMA,
    ],
)
def _sc_count(dst_hbm, cnt_out, didx, ones, acc, sem):
    cid = lax.axis_index("c")
    tid = lax.axis_index("s")

    def fill(val):
        def body(i, _):
            for j in range(HD // LANES):
                ones[i, pl.ds(j * LANES, LANES)] = jnp.full((LANES,), val, jnp.float32)
            return 0
        lax.fori_loop(0, CH2, body, 0)

    fill(0.0)
    for k in range(SLAB // CH2):
        pltpu.sync_copy(ones, acc.at[pl.ds(tid * SLAB + k * CH2, CH2)])
    fill(1.0)

    plsc.subcore_barrier()

    wid = tid * NC + cid
    ebase = wid * EPW

    def step(k, _):
        pltpu.sync_copy(dst_hbm.at[pl.ds(ebase + k * CH2, CH2)], didx)
        pltpu.sync_copy(ones, acc.at[didx], add=True)
        return 0
    lax.fori_loop(0, NCHUNK2, step, 0)

    plsc.subcore_barrier()

    for k in range(SLAB // CH2):
        r0 = tid * SLAB + k * CH2
        pltpu.sync_copy(acc.at[pl.ds(r0, CH2)], ones)
        pltpu.sync_copy(ones, cnt_out.at[pl.ds(cid * NP + r0, CH2)])


@functools.partial(
    pl.kernel,
    out_type=jax.ShapeDtypeStruct((E, DE), jnp.float32),
    mesh=_mesh,
    scratch_types=[
        pltpu.VMEM((CH2,), jnp.int32),
        pltpu.VMEM((CH2,), jnp.int32),
        pltpu.VMEM((CH2, HD), jnp.float32),
        pltpu.VMEM((CH2, HD), jnp.float32),
        pltpu.VMEM((CH2, DE), jnp.float32),
        pltpu.SemaphoreType.DMA,
    ],
)
def _sc_edge_update(t_hbm, r_hbm, src_hbm, dst_hbm, out_hbm,
                    sidx, didx, pv, qv, rv, sem):
    cid = lax.axis_index("c")
    tid = lax.axis_index("s")
    wid = tid * NC + cid
    ebase = wid * EPW

    def step(k, _):
        e0 = ebase + k * CH2
        pltpu.sync_copy(src_hbm.at[pl.ds(e0, CH2)], sidx)
        pltpu.sync_copy(dst_hbm.at[pl.ds(e0, CH2)], didx)
        pltpu.async_copy(t_hbm.at[sidx], pv, sem).wait()
        pltpu.async_copy(t_hbm.at[didx], qv, sem).wait()
        pltpu.sync_copy(r_hbm.at[pl.ds(e0, CH2)], rv)

        def rfix(i, _):
            rv[i, :] = jnp.maximum(
                pv[i, pl.ds(0, DE)] + qv[i, pl.ds(DE, DE)] + rv[i, :], 0.0)
            return 0
        lax.fori_loop(0, CH2, rfix, 0)
        pltpu.sync_copy(rv, out_hbm.at[pl.ds(e0, CH2)])
        return 0
    lax.fori_loop(0, NCHUNK2, step, 0)


_BR = 400
_NB = N // _BR
_BR2 = 1000
_NB2 = E // _BR2


def _h_body(x_ref, w_ref, o_ref):
    o_ref[...] = jnp.dot(x_ref[...], w_ref[...], preferred_element_type=jnp.float32)


def _h_tc(x, w):
    return pl.pallas_call(
        _h_body,
        grid=(_NB, 2),
        in_specs=[pl.BlockSpec((_BR, D), lambda i, j: (i, 0)),
                  pl.BlockSpec((D, HD), lambda i, j: (0, j))],
        out_specs=pl.BlockSpec((_BR, HD), lambda i, j: (j * _NB + i, 0)),
        out_shape=jax.ShapeDtypeStruct((2 * N, HD), jnp.float32),
    )(x, w)


def _c_body(ea_ref, w_ref, b_ref, o_ref):
    o_ref[...] = jnp.dot(ea_ref[...], w_ref[...],
                         preferred_element_type=jnp.float32) + b_ref[0]


def _c_tc(ea, w, b2):
    return pl.pallas_call(
        _c_body,
        grid=(_NB2, 2),
        in_specs=[pl.BlockSpec((_BR2, DE), lambda i, j: (i, 0)),
                  pl.BlockSpec((DE, HD), lambda i, j: (0, j)),
                  pl.BlockSpec((1, 1, HD), lambda i, j: (j, 0, 0))],
        out_specs=pl.BlockSpec((_BR2, HD), lambda i, j: (j * _NB2 + i, 0)),
        out_shape=jax.ShapeDtypeStruct((2 * E, HD), jnp.float32),
    )(ea, w, b2)


def _upd_body(s0_ref, s1_ref, cnt0_ref, cnt1_ref, x_ref,
              wm0_ref, wm1_ref, wx_ref, b_ref, o_ref):
    c = jnp.maximum(cnt0_ref[0, :, 0:1] + cnt1_ref[0, :, 0:1], 1.0)
    acc = jnp.dot(s0_ref[0] / c, wm0_ref[...], preferred_element_type=jnp.float32)
    acc = acc + jnp.dot(s1_ref[0] / c, wm1_ref[...], preferred_element_type=jnp.float32)
    acc = acc + jnp.dot(x_ref[...], wx_ref[...], preferred_element_type=jnp.float32)
    o_ref[...] = jnp.maximum(acc + b_ref[...], 0.0)


def _upd_tc(S3, CNT3, x, Wa, ba):
    wm0 = Wa[:HD]
    wm1 = Wa[HD:D]
    wx = Wa[D:]
    return pl.pallas_call(
        _upd_body,
        grid=(_NB,),
        in_specs=[pl.BlockSpec((1, _BR, HD), lambda i: (0, i, 0)),
                  pl.BlockSpec((1, _BR, HD), lambda i: (1, i, 0)),
                  pl.BlockSpec((1, _BR, HD), lambda i: (0, i, 0)),
                  pl.BlockSpec((1, _BR, HD), lambda i: (1, i, 0)),
                  pl.BlockSpec((_BR, D), lambda i: (i, 0)),
                  pl.BlockSpec((HD, D), lambda i: (0, 0)),
                  pl.BlockSpec((HD, D), lambda i: (0, 0)),
                  pl.BlockSpec((D, D), lambda i: (0, 0)),
                  pl.BlockSpec((1, D), lambda i: (0, 0))],
        out_specs=pl.BlockSpec((_BR, D), lambda i: (i, 0)),
        out_shape=jax.ShapeDtypeStruct((N, D), jnp.float32),
    )(S3, S3, CNT3, CNT3, x, wm0, wm1, wx, ba.reshape(1, D))


def _pq_body(x_ref, w_ref, t_ref):
    t_ref[...] = jnp.dot(x_ref[...], w_ref[...],
                         preferred_element_type=jnp.float32)


def _pq_tc(x, ws, wd):
    w = jnp.concatenate([ws, wd, jnp.zeros((D, HD - 2 * DE), jnp.float32)], axis=1)
    return pl.pallas_call(
        _pq_body,
        grid=(_NB,),
        in_specs=[pl.BlockSpec((_BR, D), lambda i: (i, 0)),
                  pl.BlockSpec((D, HD), lambda i: (0, 0))],
        out_specs=pl.BlockSpec((_BR, HD), lambda i: (i, 0)),
        out_shape=jax.ShapeDtypeStruct((N, HD), jnp.float32),
    )(x, w)


def _r_body(ea_ref, w_ref, b_ref, o_ref):
    o_ref[...] = jnp.dot(ea_ref[...], w_ref[...],
                         preferred_element_type=jnp.float32) + b_ref[...]


def _r_tc(ea, w, b):
    return pl.pallas_call(
        _r_body,
        grid=(_NB2,),
        in_specs=[pl.BlockSpec((_BR2, DE), lambda i: (i, 0)),
                  pl.BlockSpec((DE, DE), lambda i: (0, 0)),
                  pl.BlockSpec((1, DE), lambda i: (0, 0))],
        out_specs=pl.BlockSpec((_BR2, DE), lambda i: (i, 0)),
        out_shape=jax.ShapeDtypeStruct((E, DE), jnp.float32),
    )(ea, w, b.reshape(1, DE))


def kernel(x, edge_attr, edge_index, Wm0, bm0, Wa0, ba0, Wm1, bm1, Wa1, ba1, We0, be0):
    src = edge_index[0].astype(jnp.int32)
    dst = edge_index[1].astype(jnp.int32)
    src2 = jnp.concatenate([src, src + N])

    CNT3 = _sc_count(dst).reshape(2, NP, HD)

    Hc = _h_tc(x, Wm0[:D])
    Cc = _c_tc(edge_attr, Wm0[D:], bm0.reshape(2, 1, HD))
    S0 = _sc_scatter(Hc, Cc, src2, dst)
    x1 = _upd_tc(S0.reshape(2, NP, HD), CNT3, x, Wa0, ba0)

    T = _pq_tc(x1, We0[:D], We0[D:2 * D])
    R = _r_tc(edge_attr, We0[2 * D:], be0)
    ea1 = _sc_edge_update(T, R, src, dst)

    H2 = _h_tc(x1, Wm1[:D])
    C2 = _c_tc(ea1, Wm1[D:], bm1.reshape(2, 1, HD))
    S1 = _sc_scatter(H2, C2, src2, dst)
    x2 = _upd_tc(S1.reshape(2, NP, HD), CNT3, x1, Wa1, ba1)
    return x2


# speedup vs baseline: 1.7320x; 1.3454x over previous
import functools

import jax
import jax.numpy as jnp
from jax import lax
from jax.experimental import pallas as pl
from jax.experimental.pallas import tpu as pltpu
from jax.experimental.pallas import tpu_sc as plsc

N = 10000
E = 160000
D = 256
DE = 16
HD = 128

NC = 2
NS = 16
LANES = 16

NP = 10240
SLAB = NP // NS

EPT = E // NS
CH = 80
NCHUNK = EPT // CH

NW = NC * NS
EPW = E // NW
CH2 = 40
NCHUNK2 = EPW // CH2

_mesh = plsc.VectorSubcoreMesh(core_axis_name="c", subcore_axis_name="s")


@functools.partial(
    pl.kernel,
    out_type=jax.ShapeDtypeStruct((2 * NP, HD), jnp.float32),
    mesh=_mesh,
    scratch_types=[
        pltpu.VMEM((CH,), jnp.int32),
        pltpu.VMEM((CH,), jnp.int32),
        pltpu.VMEM((CH,), jnp.int32),
        pltpu.VMEM((CH,), jnp.int32),
        pltpu.VMEM((CH, HD), jnp.float32),
        pltpu.VMEM((CH, HD), jnp.float32),
        pltpu.VMEM((CH, HD), jnp.float32),
        pltpu.VMEM((CH, HD), jnp.float32),
        pltpu.VMEM_SHARED((NP, HD), jnp.float32),
        pltpu.SemaphoreType.DMA,
        pltpu.SemaphoreType.DMA,
        pltpu.SemaphoreType.DMA,
        pltpu.SemaphoreType.DMA,
    ],
)
def _sc_scatter(h_hbm, c_hbm, src_hbm, dst_hbm, s_out,
                sidx0, sidx1, didx0, didx1, rows0, rows1, crows0, crows1,
                acc, semg0, semg1, semi0, semi1):
    cid = lax.axis_index("c")
    tid = lax.axis_index("s")
    sidx = (sidx0, sidx1)
    didx = (didx0, didx1)
    rows = (rows0, rows1)
    crows = (crows0, crows1)
    semg = (semg0, semg1)
    semi = (semi0, semi1)

    def zrow(i, _):
        for j in range(HD // LANES):
            rows0[i, pl.ds(j * LANES, LANES)] = jnp.zeros((LANES,), jnp.float32)
        return 0
    lax.fori_loop(0, CH, zrow, 0)
    for k in range(SLAB // CH):
        pltpu.sync_copy(rows0, acc.at[pl.ds(tid * SLAB + k * CH, CH)])

    plsc.subcore_barrier()

    ebase = tid * EPT

    def fire_idx(e0, p):
        pltpu.async_copy(src_hbm.at[pl.ds(cid * E + e0, CH)], sidx[p], semi[p])
        pltpu.async_copy(dst_hbm.at[pl.ds(e0, CH)], didx[p], semi[p])

    def wait_idx(p):
        pltpu.make_async_copy(src_hbm.at[pl.ds(0, CH)], sidx[p], semi[p]).wait()
        pltpu.make_async_copy(dst_hbm.at[pl.ds(0, CH)], didx[p], semi[p]).wait()

    def fire_data(e0, p):
        pltpu.async_copy(h_hbm.at[sidx[p]], rows[p], semg[p])
        pltpu.async_copy(c_hbm.at[pl.ds(cid * E + e0, CH)], crows[p], semg[p])

    def wait_data(p):
        pltpu.make_async_copy(h_hbm.at[sidx[p]], rows[p], semg[p]).wait()
        pltpu.make_async_copy(c_hbm.at[pl.ds(0, CH)], crows[p], semg[p]).wait()

    def combine_scatter(p):
        def rfix(i, _):
            for j in range(HD // LANES):
                sl = pl.ds(j * LANES, LANES)
                rows[p][i, sl] = jnp.maximum(rows[p][i, sl] + crows[p][i, sl], 0.0)
            return 0
        lax.fori_loop(0, CH, rfix, 0)
        pltpu.sync_copy(rows[p], acc.at[didx[p]], add=True)

    pltpu.sync_copy(src_hbm.at[pl.ds(cid * E + ebase, CH)], sidx0)
    pltpu.sync_copy(dst_hbm.at[pl.ds(ebase, CH)], didx0)
    fire_data(ebase, 0)
    fire_idx(ebase + CH, 1)

    def outer(jj, _):
        k0 = 2 * jj
        for b in (0, 1):
            k = k0 + b
            nxt = 1 - b
            wait_idx(nxt)
            fire_data(ebase + (k + 1) * CH, nxt)
            wait_data(b)
            combine_scatter(b)
            k2 = jnp.minimum(k + 2, NCHUNK - 1)
            fire_idx(ebase + k2 * CH, b)
        return 0
    lax.fori_loop(0, (NCHUNK - 1) // 2, outer, 0)

    wait_data(0)
    combine_scatter(0)
    wait_idx(1)

    plsc.subcore_barrier()

    for k in range(SLAB // CH):
        r0 = tid * SLAB + k * CH
        pltpu.sync_copy(acc.at[pl.ds(r0, CH)], rows0)
        pltpu.sync_copy(rows0, s_out.at[pl.ds(cid * NP + r0, CH)])


@functools.partial(
    pl.kernel,
    out_type=jax.ShapeDtypeStruct((2 * NP, HD), jnp.float32),
    mesh=_mesh,
    scratch_types=[
        pltpu.VMEM((CH2,), jnp.int32),
        pltpu.VMEM((CH2, HD), jnp.float32),
        pltpu.VMEM_SHARED((NP, HD), jnp.float32),
        pltpu.SemaphoreType.DMA,
    ],
)
def _sc_count(dst_hbm, cnt_out, didx, ones, acc, sem):
    cid = lax.axis_index("c")
    tid = lax.axis_index("s")

    def fill(val):
        def body(i, _):
            for j in range(HD // LANES):
                ones[i, pl.ds(j * LANES, LANES)] = jnp.full((LANES,), val, jnp.float32)
            return 0
        lax.fori_loop(0, CH2, body, 0)

    fill(0.0)
    for k in range(SLAB // CH2):
        pltpu.sync_copy(ones, acc.at[pl.ds(tid * SLAB + k * CH2, CH2)])
    fill(1.0)

    plsc.subcore_barrier()

    wid = tid * NC + cid
    ebase = wid * EPW

    def step(k, _):
        pltpu.sync_copy(dst_hbm.at[pl.ds(ebase + k * CH2, CH2)], didx)
        pltpu.sync_copy(ones, acc.at[didx], add=True)
        return 0
    lax.fori_loop(0, NCHUNK2, step, 0)

    plsc.subcore_barrier()

    for k in range(SLAB // CH2):
        r0 = tid * SLAB + k * CH2
        pltpu.sync_copy(acc.at[pl.ds(r0, CH2)], ones)
        pltpu.sync_copy(ones, cnt_out.at[pl.ds(cid * NP + r0, CH2)])


@functools.partial(
    pl.kernel,
    out_type=jax.ShapeDtypeStruct((E, DE), jnp.float32),
    mesh=_mesh,
    scratch_types=[
        pltpu.VMEM((CH2,), jnp.int32),
        pltpu.VMEM((CH2,), jnp.int32),
        pltpu.VMEM((CH2, HD), jnp.float32),
        pltpu.VMEM((CH2, HD), jnp.float32),
        pltpu.VMEM((CH2, DE), jnp.float32),
        pltpu.SemaphoreType.DMA,
    ],
)
def _sc_edge_update(t_hbm, r_hbm, src_hbm, dst_hbm, out_hbm,
                    sidx, didx, pv, qv, rv, sem):
    cid = lax.axis_index("c")
    tid = lax.axis_index("s")
    wid = tid * NC + cid
    ebase = wid * EPW

    def step(k, _):
        e0 = ebase + k * CH2
        pltpu.sync_copy(src_hbm.at[pl.ds(e0, CH2)], sidx)
        pltpu.sync_copy(dst_hbm.at[pl.ds(e0, CH2)], didx)
        pltpu.async_copy(t_hbm.at[sidx], pv, sem).wait()
        pltpu.async_copy(t_hbm.at[didx], qv, sem).wait()
        pltpu.sync_copy(r_hbm.at[pl.ds(e0, CH2)], rv)

        def rfix(i, _):
            rv[i, :] = jnp.maximum(
                pv[i, pl.ds(0, DE)] + qv[i, pl.ds(DE, DE)] + rv[i, :], 0.0)
            return 0
        lax.fori_loop(0, CH2, rfix, 0)
        pltpu.sync_copy(rv, out_hbm.at[pl.ds(e0, CH2)])
        return 0
    lax.fori_loop(0, NCHUNK2, step, 0)


_BR = 400
_NB = N // _BR
_BR2 = 1000
_NB2 = E // _BR2


def _h_body(x_ref, w_ref, o_ref):
    o_ref[...] = jnp.dot(x_ref[...], w_ref[...], preferred_element_type=jnp.float32)


def _h_tc(x, w):
    return pl.pallas_call(
        _h_body,
        grid=(_NB, 2),
        in_specs=[pl.BlockSpec((_BR, D), lambda i, j: (i, 0)),
                  pl.BlockSpec((D, HD), lambda i, j: (0, j))],
        out_specs=pl.BlockSpec((_BR, HD), lambda i, j: (j * _NB + i, 0)),
        out_shape=jax.ShapeDtypeStruct((2 * N, HD), jnp.float32),
    )(x, w)


def _c_body(ea_ref, w_ref, b_ref, o_ref):
    o_ref[...] = jnp.dot(ea_ref[...], w_ref[...],
                         preferred_element_type=jnp.float32) + b_ref[0]


def _c_tc(ea, w, b2):
    return pl.pallas_call(
        _c_body,
        grid=(_NB2, 2),
        in_specs=[pl.BlockSpec((_BR2, DE), lambda i, j: (i, 0)),
                  pl.BlockSpec((DE, HD), lambda i, j: (0, j)),
                  pl.BlockSpec((1, 1, HD), lambda i, j: (j, 0, 0))],
        out_specs=pl.BlockSpec((_BR2, HD), lambda i, j: (j * _NB2 + i, 0)),
        out_shape=jax.ShapeDtypeStruct((2 * E, HD), jnp.float32),
    )(ea, w, b2)


def _upd_body(s0_ref, s1_ref, cnt0_ref, cnt1_ref, x_ref,
              wm0_ref, wm1_ref, wx_ref, b_ref, o_ref):
    c = jnp.maximum(cnt0_ref[0, :, 0:1] + cnt1_ref[0, :, 0:1], 1.0)
    acc = jnp.dot(s0_ref[0] / c, wm0_ref[...], preferred_element_type=jnp.float32)
    acc = acc + jnp.dot(s1_ref[0] / c, wm1_ref[...], preferred_element_type=jnp.float32)
    acc = acc + jnp.dot(x_ref[...], wx_ref[...], preferred_element_type=jnp.float32)
    o_ref[...] = jnp.maximum(acc + b_ref[...], 0.0)


def _upd_tc(S3, CNT3, x, Wa, ba):
    wm0 = Wa[:HD]
    wm1 = Wa[HD:D]
    wx = Wa[D:]
    return pl.pallas_call(
        _upd_body,
        grid=(_NB,),
        in_specs=[pl.BlockSpec((1, _BR, HD), lambda i: (0, i, 0)),
                  pl.BlockSpec((1, _BR, HD), lambda i: (1, i, 0)),
                  pl.BlockSpec((1, _BR, HD), lambda i: (0, i, 0)),
                  pl.BlockSpec((1, _BR, HD), lambda i: (1, i, 0)),
                  pl.BlockSpec((_BR, D), lambda i: (i, 0)),
                  pl.BlockSpec((HD, D), lambda i: (0, 0)),
                  pl.BlockSpec((HD, D), lambda i: (0, 0)),
                  pl.BlockSpec((D, D), lambda i: (0, 0)),
                  pl.BlockSpec((1, D), lambda i: (0, 0))],
        out_specs=pl.BlockSpec((_BR, D), lambda i: (i, 0)),
        out_shape=jax.ShapeDtypeStruct((N, D), jnp.float32),
    )(S3, S3, CNT3, CNT3, x, wm0, wm1, wx, ba.reshape(1, D))


def _pq_body(x_ref, w_ref, t_ref):
    t_ref[...] = jnp.dot(x_ref[...], w_ref[...],
                         preferred_element_type=jnp.float32)


def _pq_tc(x, ws, wd):
    w = jnp.concatenate([ws, wd, jnp.zeros((D, HD - 2 * DE), jnp.float32)], axis=1)
    return pl.pallas_call(
        _pq_body,
        grid=(_NB,),
        in_specs=[pl.BlockSpec((_BR, D), lambda i: (i, 0)),
                  pl.BlockSpec((D, HD), lambda i: (0, 0))],
        out_specs=pl.BlockSpec((_BR, HD), lambda i: (i, 0)),
        out_shape=jax.ShapeDtypeStruct((N, HD), jnp.float32),
    )(x, w)


def _r_body(ea_ref, w_ref, b_ref, o_ref):
    o_ref[...] = jnp.dot(ea_ref[...], w_ref[...],
                         preferred_element_type=jnp.float32) + b_ref[...]


def _r_tc(ea, w, b):
    return pl.pallas_call(
        _r_body,
        grid=(_NB2,),
        in_specs=[pl.BlockSpec((_BR2, DE), lambda i: (i, 0)),
                  pl.BlockSpec((DE, DE), lambda i: (0, 0)),
                  pl.BlockSpec((1, DE), lambda i: (0, 0))],
        out_specs=pl.BlockSpec((_BR2, DE), lambda i: (i, 0)),
        out_shape=jax.ShapeDtypeStruct((E, DE), jnp.float32),
    )(ea, w, b.reshape(1, DE))


def kernel(x, edge_attr, edge_index, Wm0, bm0, Wa0, ba0, Wm1, bm1, Wa1, ba1, We0, be0):
    src = edge_index[0].astype(jnp.int32)
    dst = edge_index[1].astype(jnp.int32)
    src2 = jnp.concatenate([src, src + N])

    CNT3 = _sc_count(dst).reshape(2, NP, HD)

    Hc = _h_tc(x, Wm0[:D])
    Cc = _c_tc(edge_attr, Wm0[D:], bm0.reshape(2, 1, HD))
    S0 = _sc_scatter(Hc, Cc, src2, dst)
    x1 = _upd_tc(S0.reshape(2, NP, HD), CNT3, x, Wa0, ba0)

    T = _pq_tc(x1, We0[:D], We0[D:2 * D])
    R = _r_tc(edge_attr, We0[2 * D:], be0)
    ea1 = _sc_edge_update(T, R, src, dst)

    H2 = _h_tc(x1, Wm1[:D])
    C2 = _c_tc(ea1, Wm1[D:], bm1.reshape(2, 1, HD))
    S1 = _sc_scatter(H2, C2, src2, dst)
    x2 = _upd_tc(S1.reshape(2, NP, HD), CNT3, x1, Wa1, ba1)
    return x2


# speedup vs baseline: 2.1809x; 1.2591x over previous
import functools

import jax
import jax.numpy as jnp
from jax import lax
from jax.experimental import pallas as pl
from jax.experimental.pallas import tpu as pltpu
from jax.experimental.pallas import tpu_sc as plsc

N = 10000
E = 160000
D = 256
DE = 16
HD = 128

NC = 2
NS = 16
LANES = 16

NP = 10240
SLAB = NP // NS

EPT = E // NS
CH = 80
NCHUNK = EPT // CH

NW = NC * NS
EPW = E // NW
CH2 = 40
NCHUNK2 = EPW // CH2

_mesh = plsc.VectorSubcoreMesh(core_axis_name="c", subcore_axis_name="s")


@functools.partial(
    pl.kernel,
    out_type=jax.ShapeDtypeStruct((2 * NP, HD), jnp.float32),
    mesh=_mesh,
    scratch_types=[
        pltpu.VMEM((CH,), jnp.int32),
        pltpu.VMEM((CH,), jnp.int32),
        pltpu.VMEM((CH,), jnp.int32),
        pltpu.VMEM((CH,), jnp.int32),
        pltpu.VMEM((CH, HD), jnp.float32),
        pltpu.VMEM((CH, HD), jnp.float32),
        pltpu.VMEM((CH, HD), jnp.float32),
        pltpu.VMEM((CH, HD), jnp.float32),
        pltpu.VMEM_SHARED((NP, HD), jnp.float32),
        pltpu.SemaphoreType.DMA,
        pltpu.SemaphoreType.DMA,
        pltpu.SemaphoreType.DMA,
        pltpu.SemaphoreType.DMA,
    ],
)
def _sc_scatter(h_hbm, c_hbm, src_hbm, dst_hbm, s_out,
                sidx0, sidx1, didx0, didx1, rows0, rows1, crows0, crows1,
                acc, semg0, semg1, semi0, semi1):
    cid = lax.axis_index("c")
    tid = lax.axis_index("s")
    sidx = (sidx0, sidx1)
    didx = (didx0, didx1)
    rows = (rows0, rows1)
    crows = (crows0, crows1)
    semg = (semg0, semg1)
    semi = (semi0, semi1)

    def zrow(i, _):
        for j in range(HD // LANES):
            rows0[i, pl.ds(j * LANES, LANES)] = jnp.zeros((LANES,), jnp.float32)
        return 0
    lax.fori_loop(0, CH, zrow, 0)
    for k in range(SLAB // CH):
        pltpu.sync_copy(rows0, acc.at[pl.ds(tid * SLAB + k * CH, CH)])

    plsc.subcore_barrier()

    ebase = tid * EPT

    def fire_idx(e0, p):
        pltpu.async_copy(src_hbm.at[pl.ds(cid * E + e0, CH)], sidx[p], semi[p])
        pltpu.async_copy(dst_hbm.at[pl.ds(e0, CH)], didx[p], semi[p])

    def wait_idx(p):
        pltpu.make_async_copy(src_hbm.at[pl.ds(0, CH)], sidx[p], semi[p]).wait()
        pltpu.make_async_copy(dst_hbm.at[pl.ds(0, CH)], didx[p], semi[p]).wait()

    def fire_data(e0, p):
        pltpu.async_copy(h_hbm.at[sidx[p]], rows[p], semg[p])
        pltpu.async_copy(c_hbm.at[pl.ds(cid * E + e0, CH)], crows[p], semg[p])

    def wait_data(p):
        pltpu.make_async_copy(h_hbm.at[sidx[p]], rows[p], semg[p]).wait()
        pltpu.make_async_copy(c_hbm.at[pl.ds(0, CH)], crows[p], semg[p]).wait()

    def combine_scatter(p):
        def rfix(i, _):
            for j in range(HD // LANES):
                sl = pl.ds(j * LANES, LANES)
                rows[p][i, sl] = jnp.maximum(rows[p][i, sl] + crows[p][i, sl], 0.0)
            return 0
        lax.fori_loop(0, CH, rfix, 0)
        pltpu.sync_copy(rows[p], acc.at[didx[p]], add=True)

    pltpu.sync_copy(src_hbm.at[pl.ds(cid * E + ebase, CH)], sidx0)
    pltpu.sync_copy(dst_hbm.at[pl.ds(ebase, CH)], didx0)
    fire_data(ebase, 0)
    fire_idx(ebase + CH, 1)

    def outer(jj, _):
        k0 = 2 * jj
        for b in (0, 1):
            k = k0 + b
            nxt = 1 - b
            wait_idx(nxt)
            fire_data(ebase + (k + 1) * CH, nxt)
            wait_data(b)
            combine_scatter(b)
            k2 = jnp.minimum(k + 2, NCHUNK - 1)
            fire_idx(ebase + k2 * CH, b)
        return 0
    lax.fori_loop(0, (NCHUNK - 1) // 2, outer, 0)

    wait_data(0)
    combine_scatter(0)
    wait_idx(1)

    plsc.subcore_barrier()

    for k in range(SLAB // CH):
        r0 = tid * SLAB + k * CH
        pltpu.sync_copy(acc.at[pl.ds(r0, CH)], rows0)
        pltpu.sync_copy(rows0, s_out.at[pl.ds(cid * NP + r0, CH)])


@functools.partial(
    pl.kernel,
    out_type=jax.ShapeDtypeStruct((2 * NP, HD), jnp.float32),
    mesh=_mesh,
    scratch_types=[
        pltpu.VMEM((CH2,), jnp.int32),
        pltpu.VMEM((CH2, HD), jnp.float32),
        pltpu.VMEM_SHARED((NP, HD), jnp.float32),
        pltpu.SemaphoreType.DMA,
    ],
)
def _sc_count(dst_hbm, cnt_out, didx, ones, acc, sem):
    cid = lax.axis_index("c")
    tid = lax.axis_index("s")

    def fill(val):
        def body(i, _):
            for j in range(HD // LANES):
                ones[i, pl.ds(j * LANES, LANES)] = jnp.full((LANES,), val, jnp.float32)
            return 0
        lax.fori_loop(0, CH2, body, 0)

    fill(0.0)
    for k in range(SLAB // CH2):
        pltpu.sync_copy(ones, acc.at[pl.ds(tid * SLAB + k * CH2, CH2)])
    fill(1.0)

    plsc.subcore_barrier()

    wid = tid * NC + cid
    ebase = wid * EPW

    def step(k, _):
        pltpu.sync_copy(dst_hbm.at[pl.ds(ebase + k * CH2, CH2)], didx)
        pltpu.sync_copy(ones, acc.at[didx], add=True)
        return 0
    lax.fori_loop(0, NCHUNK2, step, 0)

    plsc.subcore_barrier()

    for k in range(SLAB // CH2):
        r0 = tid * SLAB + k * CH2
        pltpu.sync_copy(acc.at[pl.ds(r0, CH2)], ones)
        pltpu.sync_copy(ones, cnt_out.at[pl.ds(cid * NP + r0, CH2)])


EPW3 = 5120
CH3 = 128


@functools.partial(
    pl.kernel,
    out_type=jax.ShapeDtypeStruct((E, DE), jnp.float32),
    mesh=_mesh,
    scratch_types=[
        pltpu.VMEM((CH3,), jnp.int32),
        pltpu.VMEM((CH3,), jnp.int32),
        pltpu.VMEM((CH3,), jnp.int32),
        pltpu.VMEM((CH3,), jnp.int32),
        pltpu.VMEM((CH3, HD), jnp.float32),
        pltpu.VMEM((CH3, HD), jnp.float32),
        pltpu.VMEM((CH3, HD), jnp.float32),
        pltpu.VMEM((CH3, HD), jnp.float32),
        pltpu.VMEM((CH3, DE), jnp.float32),
        pltpu.VMEM((CH3, DE), jnp.float32),
        pltpu.SemaphoreType.DMA,
        pltpu.SemaphoreType.DMA,
        pltpu.SemaphoreType.DMA,
        pltpu.SemaphoreType.DMA,
    ],
)
def _sc_edge_update(t_hbm, r_hbm, src_hbm, dst_hbm, out_hbm,
                    sidx0, sidx1, didx0, didx1, pv0, pv1, qv0, qv1, rv0, rv1,
                    semg0, semg1, semi0, semi1):
    cid = lax.axis_index("c")
    tid = lax.axis_index("s")
    wid = tid * NC + cid
    ebase = wid * EPW3
    nch = jnp.where(wid == NW - 1, (E - (NW - 1) * EPW3) // CH3, EPW3 // CH3)

    sidx = (sidx0, sidx1)
    didx = (didx0, didx1)
    pv = (pv0, pv1)
    qv = (qv0, qv1)
    rv = (rv0, rv1)
    semg = (semg0, semg1)
    semi = (semi0, semi1)

    def fire_idx(e0, p):
        pltpu.async_copy(src_hbm.at[pl.ds(e0, CH3)], sidx[p], semi[p])
        pltpu.async_copy(dst_hbm.at[pl.ds(e0, CH3)], didx[p], semi[p])

    def wait_idx(p):
        pltpu.make_async_copy(src_hbm.at[pl.ds(0, CH3)], sidx[p], semi[p]).wait()
        pltpu.make_async_copy(dst_hbm.at[pl.ds(0, CH3)], didx[p], semi[p]).wait()

    def fire_data(e0, p):
        pltpu.async_copy(t_hbm.at[sidx[p]], pv[p], semg[p])
        pltpu.async_copy(t_hbm.at[didx[p]], qv[p], semg[p])
        pltpu.async_copy(r_hbm.at[pl.ds(e0, CH3)], rv[p], semg[p])

    def wait_data(p):
        pltpu.make_async_copy(t_hbm.at[sidx[p]], pv[p], semg[p]).wait()
        pltpu.make_async_copy(t_hbm.at[didx[p]], qv[p], semg[p]).wait()
        pltpu.make_async_copy(r_hbm.at[pl.ds(0, CH3)], rv[p], semg[p]).wait()

    def combine_store(e0, p):
        def rfix(i, _):
            rv[p][i, :] = jnp.maximum(
                pv[p][i, pl.ds(0, DE)] + qv[p][i, pl.ds(DE, DE)] + rv[p][i, :], 0.0)
            return 0
        lax.fori_loop(0, CH3, rfix, 0)
        pltpu.sync_copy(rv[p], out_hbm.at[pl.ds(e0, CH3)])

    pltpu.sync_copy(src_hbm.at[pl.ds(ebase, CH3)], sidx0)
    pltpu.sync_copy(dst_hbm.at[pl.ds(ebase, CH3)], didx0)
    fire_data(ebase, 0)
    fire_idx(ebase + CH3, 1)

    def outer(jj, _):
        k0 = 2 * jj
        for b in (0, 1):
            k = k0 + b
            nxt = 1 - b
            wait_idx(nxt)
            fire_data(ebase + (k + 1) * CH3, nxt)
            wait_data(b)
            combine_store(ebase + k * CH3, b)
            k2 = jnp.minimum(k + 2, nch - 1)
            fire_idx(ebase + k2 * CH3, b)
        return 0
    lax.fori_loop(0, nch // 2 - 1, outer, 0)

    wait_idx(1)
    fire_data(ebase + (nch - 1) * CH3, 1)
    wait_data(0)
    combine_store(ebase + (nch - 2) * CH3, 0)
    wait_data(1)
    combine_store(ebase + (nch - 1) * CH3, 1)


_BR = 400
_NB = N // _BR
_BR2 = 1000
_NB2 = E // _BR2


def _h_body(x_ref, w_ref, o_ref):
    o_ref[...] = jnp.dot(x_ref[...], w_ref[...], preferred_element_type=jnp.float32)


def _h_tc(x, w):
    return pl.pallas_call(
        _h_body,
        grid=(_NB, 2),
        in_specs=[pl.BlockSpec((_BR, D), lambda i, j: (i, 0)),
                  pl.BlockSpec((D, HD), lambda i, j: (0, j))],
        out_specs=pl.BlockSpec((_BR, HD), lambda i, j: (j * _NB + i, 0)),
        out_shape=jax.ShapeDtypeStruct((2 * N, HD), jnp.float32),
    )(x, w)


def _c_body(ea_ref, w_ref, b_ref, o_ref):
    o_ref[...] = jnp.dot(ea_ref[...], w_ref[...],
                         preferred_element_type=jnp.float32) + b_ref[0]


def _c_tc(ea, w, b2):
    return pl.pallas_call(
        _c_body,
        grid=(_NB2, 2),
        in_specs=[pl.BlockSpec((_BR2, DE), lambda i, j: (i, 0)),
                  pl.BlockSpec((DE, HD), lambda i, j: (0, j)),
                  pl.BlockSpec((1, 1, HD), lambda i, j: (j, 0, 0))],
        out_specs=pl.BlockSpec((_BR2, HD), lambda i, j: (j * _NB2 + i, 0)),
        out_shape=jax.ShapeDtypeStruct((2 * E, HD), jnp.float32),
    )(ea, w, b2)


def _upd_body(s0_ref, s1_ref, cnt0_ref, cnt1_ref, x_ref,
              wm0_ref, wm1_ref, wx_ref, b_ref, o_ref):
    c = jnp.maximum(cnt0_ref[0, :, 0:1] + cnt1_ref[0, :, 0:1], 1.0)
    acc = jnp.dot(s0_ref[0] / c, wm0_ref[...], preferred_element_type=jnp.float32)
    acc = acc + jnp.dot(s1_ref[0] / c, wm1_ref[...], preferred_element_type=jnp.float32)
    acc = acc + jnp.dot(x_ref[...], wx_ref[...], preferred_element_type=jnp.float32)
    o_ref[...] = jnp.maximum(acc + b_ref[...], 0.0)


def _upd_tc(S3, CNT3, x, Wa, ba):
    wm0 = Wa[:HD]
    wm1 = Wa[HD:D]
    wx = Wa[D:]
    return pl.pallas_call(
        _upd_body,
        grid=(_NB,),
        in_specs=[pl.BlockSpec((1, _BR, HD), lambda i: (0, i, 0)),
                  pl.BlockSpec((1, _BR, HD), lambda i: (1, i, 0)),
                  pl.BlockSpec((1, _BR, HD), lambda i: (0, i, 0)),
                  pl.BlockSpec((1, _BR, HD), lambda i: (1, i, 0)),
                  pl.BlockSpec((_BR, D), lambda i: (i, 0)),
                  pl.BlockSpec((HD, D), lambda i: (0, 0)),
                  pl.BlockSpec((HD, D), lambda i: (0, 0)),
                  pl.BlockSpec((D, D), lambda i: (0, 0)),
                  pl.BlockSpec((1, D), lambda i: (0, 0))],
        out_specs=pl.BlockSpec((_BR, D), lambda i: (i, 0)),
        out_shape=jax.ShapeDtypeStruct((N, D), jnp.float32),
    )(S3, S3, CNT3, CNT3, x, wm0, wm1, wx, ba.reshape(1, D))


def _pq_body(x_ref, w_ref, t_ref):
    t_ref[...] = jnp.dot(x_ref[...], w_ref[...],
                         preferred_element_type=jnp.float32)


def _pq_tc(x, ws, wd):
    w = jnp.concatenate([ws, wd, jnp.zeros((D, HD - 2 * DE), jnp.float32)], axis=1)
    return pl.pallas_call(
        _pq_body,
        grid=(_NB,),
        in_specs=[pl.BlockSpec((_BR, D), lambda i: (i, 0)),
                  pl.BlockSpec((D, HD), lambda i: (0, 0))],
        out_specs=pl.BlockSpec((_BR, HD), lambda i: (i, 0)),
        out_shape=jax.ShapeDtypeStruct((N, HD), jnp.float32),
    )(x, w)


def _r_body(ea_ref, w_ref, b_ref, o_ref):
    o_ref[...] = jnp.dot(ea_ref[...], w_ref[...],
                         preferred_element_type=jnp.float32) + b_ref[...]


def _r_tc(ea, w, b):
    return pl.pallas_call(
        _r_body,
        grid=(_NB2,),
        in_specs=[pl.BlockSpec((_BR2, DE), lambda i: (i, 0)),
                  pl.BlockSpec((DE, DE), lambda i: (0, 0)),
                  pl.BlockSpec((1, DE), lambda i: (0, 0))],
        out_specs=pl.BlockSpec((_BR2, DE), lambda i: (i, 0)),
        out_shape=jax.ShapeDtypeStruct((E, DE), jnp.float32),
    )(ea, w, b.reshape(1, DE))


def kernel(x, edge_attr, edge_index, Wm0, bm0, Wa0, ba0, Wm1, bm1, Wa1, ba1, We0, be0):
    src = edge_index[0].astype(jnp.int32)
    dst = edge_index[1].astype(jnp.int32)
    src2 = jnp.concatenate([src, src + N])

    CNT3 = _sc_count(dst).reshape(2, NP, HD)

    Hc = _h_tc(x, Wm0[:D])
    Cc = _c_tc(edge_attr, Wm0[D:], bm0.reshape(2, 1, HD))
    S0 = _sc_scatter(Hc, Cc, src2, dst)
    x1 = _upd_tc(S0.reshape(2, NP, HD), CNT3, x, Wa0, ba0)

    T = _pq_tc(x1, We0[:D], We0[D:2 * D])
    R = _r_tc(edge_attr, We0[2 * D:], be0)
    ea1 = _sc_edge_update(T, R, src, dst)

    H2 = _h_tc(x1, Wm1[:D])
    C2 = _c_tc(ea1, Wm1[D:], bm1.reshape(2, 1, HD))
    S1 = _sc_scatter(H2, C2, src2, dst)
    x2 = _upd_tc(S1.reshape(2, NP, HD), CNT3, x1, Wa1, ba1)
    return x2


# speedup vs baseline: 2.2708x; 1.0412x over previous
import functools

import jax
import jax.numpy as jnp
from jax import lax
from jax.experimental import pallas as pl
from jax.experimental.pallas import tpu as pltpu
from jax.experimental.pallas import tpu_sc as plsc

N = 10000
E = 160000
D = 256
DE = 16
HD = 128

NC = 2
NS = 16
LANES = 16

NP = 10240
SLAB = NP // NS

EPT = E // NS
CH = 80
NCHUNK = EPT // CH

NW = NC * NS
EPW = E // NW
CH2 = 40
NCHUNK2 = EPW // CH2

_mesh = plsc.VectorSubcoreMesh(core_axis_name="c", subcore_axis_name="s")


@functools.partial(
    pl.kernel,
    out_type=jax.ShapeDtypeStruct((2 * NP, HD), jnp.float32),
    mesh=_mesh,
    scratch_types=[
        pltpu.VMEM((CH,), jnp.int32),
        pltpu.VMEM((CH,), jnp.int32),
        pltpu.VMEM((CH,), jnp.int32),
        pltpu.VMEM((CH,), jnp.int32),
        pltpu.VMEM((CH, HD), jnp.float32),
        pltpu.VMEM((CH, HD), jnp.float32),
        pltpu.VMEM((CH, HD), jnp.float32),
        pltpu.VMEM((CH, HD), jnp.float32),
        pltpu.VMEM_SHARED((NP, HD), jnp.float32),
        pltpu.SemaphoreType.DMA,
        pltpu.SemaphoreType.DMA,
        pltpu.SemaphoreType.DMA,
        pltpu.SemaphoreType.DMA,
    ],
)
def _sc_scatter(h_hbm, c_hbm, src_hbm, dst_hbm, s_out,
                sidx0, sidx1, didx0, didx1, rows0, rows1, crows0, crows1,
                acc, semg0, semg1, semi0, semi1):
    cid = lax.axis_index("c")
    tid = lax.axis_index("s")
    sidx = (sidx0, sidx1)
    didx = (didx0, didx1)
    rows = (rows0, rows1)
    crows = (crows0, crows1)
    semg = (semg0, semg1)
    semi = (semi0, semi1)

    def zrow(i, _):
        for j in range(HD // LANES):
            rows0[i, pl.ds(j * LANES, LANES)] = jnp.zeros((LANES,), jnp.float32)
        return 0
    lax.fori_loop(0, CH, zrow, 0)
    for k in range(SLAB // CH):
        pltpu.sync_copy(rows0, acc.at[pl.ds(tid * SLAB + k * CH, CH)])

    plsc.subcore_barrier()

    ebase = tid * EPT

    def fire_idx(e0, p):
        pltpu.async_copy(src_hbm.at[pl.ds(cid * E + e0, CH)], sidx[p], semi[p])
        pltpu.async_copy(dst_hbm.at[pl.ds(e0, CH)], didx[p], semi[p])

    def wait_idx(p):
        pltpu.make_async_copy(src_hbm.at[pl.ds(0, CH)], sidx[p], semi[p]).wait()
        pltpu.make_async_copy(dst_hbm.at[pl.ds(0, CH)], didx[p], semi[p]).wait()

    def fire_data(e0, p):
        pltpu.async_copy(h_hbm.at[sidx[p]], rows[p], semg[p])
        pltpu.async_copy(c_hbm.at[pl.ds(cid * E + e0, CH)], crows[p], semg[p])

    def wait_data(p):
        pltpu.make_async_copy(h_hbm.at[sidx[p]], rows[p], semg[p]).wait()
        pltpu.make_async_copy(c_hbm.at[pl.ds(0, CH)], crows[p], semg[p]).wait()

    def combine_scatter(p):
        def rfix(i, _):
            for j in range(HD // LANES):
                sl = pl.ds(j * LANES, LANES)
                rows[p][i, sl] = jnp.maximum(rows[p][i, sl] + crows[p][i, sl], 0.0)
            return 0
        lax.fori_loop(0, CH, rfix, 0)
        pltpu.sync_copy(rows[p], acc.at[didx[p]], add=True)

    pltpu.sync_copy(src_hbm.at[pl.ds(cid * E + ebase, CH)], sidx0)
    pltpu.sync_copy(dst_hbm.at[pl.ds(ebase, CH)], didx0)
    fire_data(ebase, 0)
    fire_idx(ebase + CH, 1)

    def outer(jj, _):
        k0 = 2 * jj
        for b in (0, 1):
            k = k0 + b
            nxt = 1 - b
            wait_idx(nxt)
            fire_data(ebase + (k + 1) * CH, nxt)
            wait_data(b)
            combine_scatter(b)
            k2 = jnp.minimum(k + 2, NCHUNK - 1)
            fire_idx(ebase + k2 * CH, b)
        return 0
    lax.fori_loop(0, (NCHUNK - 1) // 2, outer, 0)

    wait_data(0)
    combine_scatter(0)
    wait_idx(1)

    plsc.subcore_barrier()

    for k in range(SLAB // CH):
        r0 = tid * SLAB + k * CH
        pltpu.sync_copy(acc.at[pl.ds(r0, CH)], rows0)
        pltpu.sync_copy(rows0, s_out.at[pl.ds(cid * NP + r0, CH)])


@functools.partial(
    pl.kernel,
    out_type=jax.ShapeDtypeStruct((2 * NP, HD), jnp.float32),
    mesh=_mesh,
    scratch_types=[
        pltpu.VMEM((128,), jnp.int32),
        pltpu.VMEM((128,), jnp.int32),
        pltpu.VMEM((128, HD), jnp.float32),
        pltpu.VMEM_SHARED((NP, HD), jnp.float32),
        pltpu.SemaphoreType.DMA,
        pltpu.SemaphoreType.DMA,
    ],
)
def _sc_count(dst_hbm, cnt_out, didx0, didx1, ones, acc, semi0, semi1):
    cid = lax.axis_index("c")
    tid = lax.axis_index("s")
    didx = (didx0, didx1)
    semi = (semi0, semi1)

    def fill(val):
        def body(i, _):
            for j in range(HD // LANES):
                ones[i, pl.ds(j * LANES, LANES)] = jnp.full((LANES,), val, jnp.float32)
            return 0
        lax.fori_loop(0, 128, body, 0)

    fill(0.0)
    for k in range(SLAB // 128):
        pltpu.sync_copy(ones, acc.at[pl.ds(tid * SLAB + k * 128, 128)])
    fill(1.0)

    plsc.subcore_barrier()

    wid = tid * NC + cid
    ebase = wid * EPW3
    nch = jnp.where(wid == NW - 1, (E - (NW - 1) * EPW3) // CH3, EPW3 // CH3)

    def fire_idx(e0, p):
        pltpu.async_copy(dst_hbm.at[pl.ds(e0, CH3)], didx[p], semi[p])

    def wait_idx(p):
        pltpu.make_async_copy(dst_hbm.at[pl.ds(0, CH3)], didx[p], semi[p]).wait()

    fire_idx(ebase, 0)

    def outer(jj, _):
        k0 = 2 * jj
        for b in (0, 1):
            k = k0 + b
            nxt = 1 - b
            wait_idx(b)
            k1 = jnp.minimum(k + 1, nch - 1)
            fire_idx(ebase + k1 * CH3, nxt)
            pltpu.sync_copy(ones, acc.at[didx[b]], add=True)
        return 0
    lax.fori_loop(0, nch // 2, outer, 0)
    wait_idx(0)

    plsc.subcore_barrier()

    for k in range(SLAB // 128):
        r0 = tid * SLAB + k * 128
        pltpu.sync_copy(acc.at[pl.ds(r0, 128)], ones)
        pltpu.sync_copy(ones, cnt_out.at[pl.ds(cid * NP + r0, 128)])


EPW3 = 5120
CH3 = 128


@functools.partial(
    pl.kernel,
    out_type=jax.ShapeDtypeStruct((E, DE), jnp.float32),
    mesh=_mesh,
    scratch_types=[
        pltpu.VMEM((CH3,), jnp.int32),
        pltpu.VMEM((CH3,), jnp.int32),
        pltpu.VMEM((CH3,), jnp.int32),
        pltpu.VMEM((CH3,), jnp.int32),
        pltpu.VMEM((CH3, HD), jnp.float32),
        pltpu.VMEM((CH3, HD), jnp.float32),
        pltpu.VMEM((CH3, HD), jnp.float32),
        pltpu.VMEM((CH3, HD), jnp.float32),
        pltpu.VMEM((CH3, DE), jnp.float32),
        pltpu.VMEM((CH3, DE), jnp.float32),
        pltpu.SemaphoreType.DMA,
        pltpu.SemaphoreType.DMA,
        pltpu.SemaphoreType.DMA,
        pltpu.SemaphoreType.DMA,
    ],
)
def _sc_edge_update(t_hbm, r_hbm, src_hbm, dst_hbm, out_hbm,
                    sidx0, sidx1, didx0, didx1, pv0, pv1, qv0, qv1, rv0, rv1,
                    semg0, semg1, semi0, semi1):
    cid = lax.axis_index("c")
    tid = lax.axis_index("s")
    wid = tid * NC + cid
    ebase = wid * EPW3
    nch = jnp.where(wid == NW - 1, (E - (NW - 1) * EPW3) // CH3, EPW3 // CH3)

    sidx = (sidx0, sidx1)
    didx = (didx0, didx1)
    pv = (pv0, pv1)
    qv = (qv0, qv1)
    rv = (rv0, rv1)
    semg = (semg0, semg1)
    semi = (semi0, semi1)

    def fire_idx(e0, p):
        pltpu.async_copy(src_hbm.at[pl.ds(e0, CH3)], sidx[p], semi[p])
        pltpu.async_copy(dst_hbm.at[pl.ds(e0, CH3)], didx[p], semi[p])

    def wait_idx(p):
        pltpu.make_async_copy(src_hbm.at[pl.ds(0, CH3)], sidx[p], semi[p]).wait()
        pltpu.make_async_copy(dst_hbm.at[pl.ds(0, CH3)], didx[p], semi[p]).wait()

    def fire_data(e0, p):
        pltpu.async_copy(t_hbm.at[sidx[p]], pv[p], semg[p])
        pltpu.async_copy(t_hbm.at[didx[p]], qv[p], semg[p])
        pltpu.async_copy(r_hbm.at[pl.ds(e0, CH3)], rv[p], semg[p])

    def wait_data(p):
        pltpu.make_async_copy(t_hbm.at[sidx[p]], pv[p], semg[p]).wait()
        pltpu.make_async_copy(t_hbm.at[didx[p]], qv[p], semg[p]).wait()
        pltpu.make_async_copy(r_hbm.at[pl.ds(0, CH3)], rv[p], semg[p]).wait()

    def combine_store(e0, p):
        def rfix(i, _):
            rv[p][i, :] = jnp.maximum(
                pv[p][i, pl.ds(0, DE)] + qv[p][i, pl.ds(DE, DE)] + rv[p][i, :], 0.0)
            return 0
        lax.fori_loop(0, CH3, rfix, 0)
        pltpu.sync_copy(rv[p], out_hbm.at[pl.ds(e0, CH3)])

    pltpu.sync_copy(src_hbm.at[pl.ds(ebase, CH3)], sidx0)
    pltpu.sync_copy(dst_hbm.at[pl.ds(ebase, CH3)], didx0)
    fire_data(ebase, 0)
    fire_idx(ebase + CH3, 1)

    def outer(jj, _):
        k0 = 2 * jj
        for b in (0, 1):
            k = k0 + b
            nxt = 1 - b
            wait_idx(nxt)
            fire_data(ebase + (k + 1) * CH3, nxt)
            wait_data(b)
            combine_store(ebase + k * CH3, b)
            k2 = jnp.minimum(k + 2, nch - 1)
            fire_idx(ebase + k2 * CH3, b)
        return 0
    lax.fori_loop(0, nch // 2 - 1, outer, 0)

    wait_idx(1)
    fire_data(ebase + (nch - 1) * CH3, 1)
    wait_data(0)
    combine_store(ebase + (nch - 2) * CH3, 0)
    wait_data(1)
    combine_store(ebase + (nch - 1) * CH3, 1)


_BR = 400
_NB = N // _BR
_BR2 = 1000
_NB2 = E // _BR2


def _h_body(x_ref, w_ref, o_ref):
    o_ref[...] = jnp.dot(x_ref[...], w_ref[...], preferred_element_type=jnp.float32)


def _h_tc(x, w):
    return pl.pallas_call(
        _h_body,
        grid=(_NB, 2),
        in_specs=[pl.BlockSpec((_BR, D), lambda i, j: (i, 0)),
                  pl.BlockSpec((D, HD), lambda i, j: (0, j))],
        out_specs=pl.BlockSpec((_BR, HD), lambda i, j: (j * _NB + i, 0)),
        out_shape=jax.ShapeDtypeStruct((2 * N, HD), jnp.float32),
    )(x, w)


def _c_body(ea_ref, w_ref, b_ref, o_ref):
    o_ref[...] = jnp.dot(ea_ref[...], w_ref[...],
                         preferred_element_type=jnp.float32) + b_ref[0]


def _c_tc(ea, w, b2):
    return pl.pallas_call(
        _c_body,
        grid=(_NB2, 2),
        in_specs=[pl.BlockSpec((_BR2, DE), lambda i, j: (i, 0)),
                  pl.BlockSpec((DE, HD), lambda i, j: (0, j)),
                  pl.BlockSpec((1, 1, HD), lambda i, j: (j, 0, 0))],
        out_specs=pl.BlockSpec((_BR2, HD), lambda i, j: (j * _NB2 + i, 0)),
        out_shape=jax.ShapeDtypeStruct((2 * E, HD), jnp.float32),
    )(ea, w, b2)


def _upd_body(s0_ref, s1_ref, cnt0_ref, cnt1_ref, x_ref,
              wm0_ref, wm1_ref, wx_ref, b_ref, o_ref):
    c = jnp.maximum(cnt0_ref[0, :, 0:1] + cnt1_ref[0, :, 0:1], 1.0)
    acc = jnp.dot(s0_ref[0] / c, wm0_ref[...], preferred_element_type=jnp.float32)
    acc = acc + jnp.dot(s1_ref[0] / c, wm1_ref[...], preferred_element_type=jnp.float32)
    acc = acc + jnp.dot(x_ref[...], wx_ref[...], preferred_element_type=jnp.float32)
    o_ref[...] = jnp.maximum(acc + b_ref[...], 0.0)


def _upd_tc(S3, CNT3, x, Wa, ba):
    wm0 = Wa[:HD]
    wm1 = Wa[HD:D]
    wx = Wa[D:]
    return pl.pallas_call(
        _upd_body,
        grid=(_NB,),
        in_specs=[pl.BlockSpec((1, _BR, HD), lambda i: (0, i, 0)),
                  pl.BlockSpec((1, _BR, HD), lambda i: (1, i, 0)),
                  pl.BlockSpec((1, _BR, HD), lambda i: (0, i, 0)),
                  pl.BlockSpec((1, _BR, HD), lambda i: (1, i, 0)),
                  pl.BlockSpec((_BR, D), lambda i: (i, 0)),
                  pl.BlockSpec((HD, D), lambda i: (0, 0)),
                  pl.BlockSpec((HD, D), lambda i: (0, 0)),
                  pl.BlockSpec((D, D), lambda i: (0, 0)),
                  pl.BlockSpec((1, D), lambda i: (0, 0))],
        out_specs=pl.BlockSpec((_BR, D), lambda i: (i, 0)),
        out_shape=jax.ShapeDtypeStruct((N, D), jnp.float32),
    )(S3, S3, CNT3, CNT3, x, wm0, wm1, wx, ba.reshape(1, D))


def _pq_body(x_ref, w_ref, t_ref):
    t_ref[...] = jnp.dot(x_ref[...], w_ref[...],
                         preferred_element_type=jnp.float32)


def _pq_tc(x, ws, wd):
    w = jnp.concatenate([ws, wd, jnp.zeros((D, HD - 2 * DE), jnp.float32)], axis=1)
    return pl.pallas_call(
        _pq_body,
        grid=(_NB,),
        in_specs=[pl.BlockSpec((_BR, D), lambda i: (i, 0)),
                  pl.BlockSpec((D, HD), lambda i: (0, 0))],
        out_specs=pl.BlockSpec((_BR, HD), lambda i: (i, 0)),
        out_shape=jax.ShapeDtypeStruct((N, HD), jnp.float32),
    )(x, w)


def _r_body(ea_ref, w_ref, b_ref, o_ref):
    o_ref[...] = jnp.dot(ea_ref[...], w_ref[...],
                         preferred_element_type=jnp.float32) + b_ref[...]


def _r_tc(ea, w, b):
    return pl.pallas_call(
        _r_body,
        grid=(_NB2,),
        in_specs=[pl.BlockSpec((_BR2, DE), lambda i: (i, 0)),
                  pl.BlockSpec((DE, DE), lambda i: (0, 0)),
                  pl.BlockSpec((1, DE), lambda i: (0, 0))],
        out_specs=pl.BlockSpec((_BR2, DE), lambda i: (i, 0)),
        out_shape=jax.ShapeDtypeStruct((E, DE), jnp.float32),
    )(ea, w, b.reshape(1, DE))


def kernel(x, edge_attr, edge_index, Wm0, bm0, Wa0, ba0, Wm1, bm1, Wa1, ba1, We0, be0):
    src = edge_index[0].astype(jnp.int32)
    dst = edge_index[1].astype(jnp.int32)
    src2 = jnp.concatenate([src, src + N])

    CNT3 = _sc_count(dst).reshape(2, NP, HD)

    Hc = _h_tc(x, Wm0[:D])
    Cc = _c_tc(edge_attr, Wm0[D:], bm0.reshape(2, 1, HD))
    S0 = _sc_scatter(Hc, Cc, src2, dst)
    x1 = _upd_tc(S0.reshape(2, NP, HD), CNT3, x, Wa0, ba0)

    T = _pq_tc(x1, We0[:D], We0[D:2 * D])
    R = _r_tc(edge_attr, We0[2 * D:], be0)
    ea1 = _sc_edge_update(T, R, src, dst)

    H2 = _h_tc(x1, Wm1[:D])
    C2 = _c_tc(ea1, Wm1[D:], bm1.reshape(2, 1, HD))
    S1 = _sc_scatter(H2, C2, src2, dst)
    x2 = _upd_tc(S1.reshape(2, NP, HD), CNT3, x1, Wa1, ba1)
    return x2
